# baseline (speedup 1.0000x reference)
import jax
import jax.numpy as jnp
from jax import lax
from jax.experimental import pallas as pl
from jax.experimental.pallas import tpu as pltpu
from jax.experimental.pallas import tpu_sc as plsc

_D = 64
_V = 49
_VD = _V * _D
_REP = 3137
_BATCH = 16384
_HIST = 200
_NC = 2
_NS = 16
_NW = _NC * _NS
_BB = 128
_TI = 8
_TO = 2
_NBLK = _BATCH // _BB // _NW
_NTC = _HIST // _TI
_NQ = _NBLK * _NTC


def _body(idx_hbm, table_hbm, out_hbm, stage_v, rep_v, idx_v0, idx_v1,
          rows_v0, rows_v1, idx_sem, out_sem):
    idx_bufs = (idx_v0, idx_v1)
    row_bufs = (rows_v0, rows_v1)
    wid = lax.axis_index("s") * _NC + lax.axis_index("c")
    wb0 = wid * (_NBLK * _BB)
    iota = lax.iota(jnp.int32, 16)
    lane_off = iota * _REP

    pltpu.sync_copy(table_hbm, stage_v)

    @plsc.parallel_loop(0, _VD // 16, step=1, unroll=2)
    def _build(m):
        x = stage_v[pl.ds(m * 16, 16)]
        dst = iota + m * 16
        for k in range(16):
            plsc.store_scatter(rep_v, [dst + k * _REP], x)

    def chunk_coords(q):
        blk = q // _NTC
        t0 = (q % _NTC) * _TI
        b0 = wb0 + blk * _BB
        return t0, b0

    def idx_start(q, qb):
        t0, b0 = chunk_coords(q)
        return pltpu.make_async_copy(
            idx_hbm.at[pl.ds(t0, _TI), pl.ds(b0, _BB)], idx_bufs[qb],
            idx_sem.at[qb])

    idx_start(0, 0).start()
    idx_start(1, 1).start()

    def compute(qb, h, rb):
        @plsc.parallel_loop(0, _BB // 16, step=1, unroll=4)
        def bg_step(bg):
            for tt in range(_TO):
                tl = h * _TO + tt
                idx16 = idx_bufs[qb][tl, pl.ds(bg * 16, 16)]
                pat = idx16 * _D + lane_off
                for j in range(_D):
                    x = plsc.load_gather(rep_v, [pat + j])
                    row_bufs[rb][tt, j, pl.ds(bg * 16, 16)] = x

    def q_step(q2, carry):
        for qb in range(2):
            q = q2 * 2 + qb
            t0, b0 = chunk_coords(q)
            idx_start(q, qb).wait()

            def h_step(h2, c2):
                for hh in range(2):
                    h = h2 * 2 + hh
                    rb = hh
                    dst = out_hbm.at[pl.ds(t0 + h * _TO, _TO), :,
                                     pl.ds(b0, _BB)]

                    def _wait_out():
                        pltpu.make_async_copy(row_bufs[rb], dst,
                                              out_sem.at[rb]).wait()

                    if qb == 0:
                        pl.when(jnp.logical_or(q2 >= 1, h2 >= 1))(_wait_out)
                    else:
                        _wait_out()

                    compute(qb, h, rb)
                    pltpu.async_copy(row_bufs[rb], dst, out_sem.at[rb])
                return c2

            lax.fori_loop(0, _TI // _TO // 2, h_step, 0)

            @pl.when(q2 < _NQ // 2 - 1)
            def _prefetch_idx():
                idx_start(q + 2, qb).start()
        return carry

    lax.fori_loop(0, _NQ // 2, q_step, 0)

    for rb in range(2):
        pltpu.make_async_copy(
            row_bufs[rb],
            out_hbm.at[pl.ds(0, _TO), :, pl.ds(wb0, _BB)],
            out_sem.at[rb]).wait()


_mesh = plsc.VectorSubcoreMesh(core_axis_name="c", subcore_axis_name="s")

_gather = pl.kernel(
    _body,
    out_type=jax.ShapeDtypeStruct((_HIST, _D, _BATCH), jnp.float32),
    mesh=_mesh,
    compiler_params=pltpu.CompilerParams(needs_layout_passes=False),
    scratch_types=[
        pltpu.VMEM((_VD,), jnp.float32),
        pltpu.VMEM((16 * _REP,), jnp.float32),
        pltpu.VMEM((_TI, _BB), jnp.int32),
        pltpu.VMEM((_TI, _BB), jnp.int32),
        pltpu.VMEM((_TO, _D, _BB), jnp.float32),
        pltpu.VMEM((_TO, _D, _BB), jnp.float32),
        pltpu.SemaphoreType.DMA((2,)),
        pltpu.SemaphoreType.DMA((2,)),
    ],
)


def kernel(time, table):
    idx_t = jnp.transpose(time)
    out_phys = _gather(idx_t, table.reshape(_VD))
    return jnp.transpose(out_phys, (2, 0, 1))


# speedup vs baseline: 1.5636x; 1.5636x over previous
import jax
import jax.numpy as jnp
from jax import lax
from jax.experimental import pallas as pl
from jax.experimental.pallas import tpu as pltpu
from jax.experimental.pallas import tpu_sc as plsc

_D = 64
_V = 49
_VD = _V * _D
_REP = 3137
_BATCH = 16384
_HIST = 200
_NC = 2
_NS = 16
_NW = _NC * _NS
_BB = 128
_TI = 8
_TO = 2
_NBLK = _BATCH // _BB // _NW
_NTC = _HIST // _TI
_NQ = _NBLK * _NTC


def _body(idx_hbm, table_hbm, out_hbm, stage_v, rep_v, idx_v0, idx_v1,
          rows_v0, rows_v1, idx_sem, out_sem):
    idx_bufs = (idx_v0, idx_v1)
    row_bufs = (rows_v0, rows_v1)
    wid = lax.axis_index("s") * _NC + lax.axis_index("c")
    wb0 = wid * (_NBLK * _BB)
    iota = lax.iota(jnp.int32, 16)
    lane_off = iota * _REP

    pltpu.sync_copy(table_hbm, stage_v)

    @plsc.parallel_loop(0, _VD // 16, step=1, unroll=2)
    def _build(m):
        x = stage_v[pl.ds(m * 16, 16)]
        dst = iota + m * 16
        for k in range(16):
            plsc.store_scatter(rep_v, [dst + k * _REP], x)

    def chunk_coords(q):
        blk = q // _NTC
        t0 = (q % _NTC) * _TI
        b0 = wb0 + blk * _BB
        return t0, b0

    def idx_start(q, qb):
        t0, b0 = chunk_coords(q)
        return pltpu.make_async_copy(
            idx_hbm.at[pl.ds(t0, _TI), pl.ds(b0, _BB)], idx_bufs[qb],
            idx_sem.at[qb])

    idx_start(0, 0).start()
    idx_start(1, 1).start()

    def compute(qb, h, rb):
        @plsc.parallel_loop(0, _BB // 16, step=1, unroll=1)
        def bg_step(bg):
            for tt in range(_TO):
                tl = h * _TO + tt
                idx16 = idx_bufs[qb][tl, pl.ds(bg * 16, 16)]
                pat = idx16 * _D + lane_off
                for j in range(_D):
                    x = plsc.load_gather(rep_v, [pat + j])
                    row_bufs[rb][tt, j, pl.ds(bg * 16, 16)] = x

    def q_step(q2, carry):
        for qb in range(2):
            q = q2 * 2 + qb
            t0, b0 = chunk_coords(q)
            idx_start(q, qb).wait()

            def h_step(h2, c2):
                for hh in range(2):
                    h = h2 * 2 + hh
                    rb = hh
                    dst = out_hbm.at[pl.ds(t0 + h * _TO, _TO), :,
                                     pl.ds(b0, _BB)]

                    def _wait_out():
                        pltpu.make_async_copy(row_bufs[rb], dst,
                                              out_sem.at[rb]).wait()

                    if qb == 0:
                        pl.when(jnp.logical_or(q2 >= 1, h2 >= 1))(_wait_out)
                    else:
                        _wait_out()

                    compute(qb, h, rb)
                    pltpu.async_copy(row_bufs[rb], dst, out_sem.at[rb])
                return c2

            lax.fori_loop(0, _TI // _TO // 2, h_step, 0)

            @pl.when(q2 < _NQ // 2 - 1)
            def _prefetch_idx():
                idx_start(q + 2, qb).start()
        return carry

    lax.fori_loop(0, _NQ // 2, q_step, 0)

    for rb in range(2):
        pltpu.make_async_copy(
            row_bufs[rb],
            out_hbm.at[pl.ds(0, _TO), :, pl.ds(wb0, _BB)],
            out_sem.at[rb]).wait()


_mesh = plsc.VectorSubcoreMesh(core_axis_name="c", subcore_axis_name="s")

_gather = pl.kernel(
    _body,
    out_type=jax.ShapeDtypeStruct((_HIST, _D, _BATCH), jnp.float32),
    mesh=_mesh,
    compiler_params=pltpu.CompilerParams(needs_layout_passes=False),
    scratch_types=[
        pltpu.VMEM((_VD,), jnp.float32),
        pltpu.VMEM((16 * _REP,), jnp.float32),
        pltpu.VMEM((_TI, _BB), jnp.int32),
        pltpu.VMEM((_TI, _BB), jnp.int32),
        pltpu.VMEM((_TO, _D, _BB), jnp.float32),
        pltpu.VMEM((_TO, _D, _BB), jnp.float32),
        pltpu.SemaphoreType.DMA((2,)),
        pltpu.SemaphoreType.DMA((2,)),
    ],
)


def kernel(time, table):
    idx_t = jnp.transpose(time)
    out_phys = _gather(idx_t, table.reshape(_VD))
    return jnp.transpose(out_phys, (2, 0, 1))


# speedup vs baseline: 1.7142x; 1.0963x over previous
import jax
import jax.numpy as jnp
from jax import lax
from jax.experimental import pallas as pl
from jax.experimental.pallas import tpu as pltpu
from jax.experimental.pallas import tpu_sc as plsc

_D = 64
_V = 49
_VD = _V * _D
_REP = 3137
_BATCH = 16384
_HIST = 200
_NC = 2
_NS = 16
_NW = _NC * _NS
_BB = 128
_TI = 8
_TO = 2
_NBLK = _BATCH // _BB // _NW
_NTC = _HIST // _TI
_NQ = _NBLK * _NTC


def _body(idx_hbm, table_hbm, out_hbm, stage_v, rep_v, idx_v0, idx_v1,
          rows_v0, rows_v1, idx_sem, out_sem):
    idx_bufs = (idx_v0, idx_v1)
    row_bufs = (rows_v0, rows_v1)
    wid = lax.axis_index("s") * _NC + lax.axis_index("c")
    wb0 = wid * (_NBLK * _BB)
    iota = lax.iota(jnp.int32, 16)
    lane_off = iota * _REP

    pltpu.sync_copy(table_hbm, stage_v)

    @plsc.parallel_loop(0, _VD // 16, step=1, unroll=2)
    def _build(m):
        x = stage_v[pl.ds(m * 16, 16)]
        dst = iota + m * 16
        for k in range(16):
            plsc.store_scatter(rep_v, [dst + k * _REP], x)

    def chunk_coords(q):
        blk = q // _NTC
        t0 = (q % _NTC) * _TI
        b0 = wb0 + blk * _BB
        return t0, b0

    def idx_start(q, qb):
        t0, b0 = chunk_coords(q)
        return pltpu.make_async_copy(
            idx_hbm.at[pl.ds(t0, _TI), pl.ds(b0, _BB)], idx_bufs[qb],
            idx_sem.at[qb])

    idx_start(0, 0).start()
    idx_start(1, 1).start()

    def compute(qb, h, rb):
        for tt in range(_TO):
            tl = h * _TO + tt

            @plsc.parallel_loop(0, _BB // 16, step=1, unroll=1)
            def bg_step(bg):
                idx16 = idx_bufs[qb][tl, pl.ds(bg * 16, 16)]
                pat = idx16 * _D + lane_off
                for j in range(_D):
                    x = plsc.load_gather(rep_v, [pat + j])
                    row_bufs[rb][tt, j, pl.ds(bg * 16, 16)] = x

    def q_step(q2, carry):
        for qb in range(2):
            q = q2 * 2 + qb
            t0, b0 = chunk_coords(q)
            idx_start(q, qb).wait()

            def h_step(h2, c2):
                for hh in range(2):
                    h = h2 * 2 + hh
                    rb = hh
                    dst = out_hbm.at[pl.ds(t0 + h * _TO, _TO), :,
                                     pl.ds(b0, _BB)]

                    def _wait_out():
                        pltpu.make_async_copy(row_bufs[rb], dst,
                                              out_sem.at[rb]).wait()

                    if qb == 0:
                        pl.when(jnp.logical_or(q2 >= 1, h2 >= 1))(_wait_out)
                    else:
                        _wait_out()

                    compute(qb, h, rb)
                    pltpu.async_copy(row_bufs[rb], dst, out_sem.at[rb])
                return c2

            lax.fori_loop(0, _TI // _TO // 2, h_step, 0)

            @pl.when(q2 < _NQ // 2 - 1)
            def _prefetch_idx():
                idx_start(q + 2, qb).start()
        return carry

    lax.fori_loop(0, _NQ // 2, q_step, 0)

    for rb in range(2):
        pltpu.make_async_copy(
            row_bufs[rb],
            out_hbm.at[pl.ds(0, _TO), :, pl.ds(wb0, _BB)],
            out_sem.at[rb]).wait()


_mesh = plsc.VectorSubcoreMesh(core_axis_name="c", subcore_axis_name="s")

_gather = pl.kernel(
    _body,
    out_type=jax.ShapeDtypeStruct((_HIST, _D, _BATCH), jnp.float32),
    mesh=_mesh,
    compiler_params=pltpu.CompilerParams(needs_layout_passes=False),
    scratch_types=[
        pltpu.VMEM((_VD,), jnp.float32),
        pltpu.VMEM((16 * _REP,), jnp.float32),
        pltpu.VMEM((_TI, _BB), jnp.int32),
        pltpu.VMEM((_TI, _BB), jnp.int32),
        pltpu.VMEM((_TO, _D, _BB), jnp.float32),
        pltpu.VMEM((_TO, _D, _BB), jnp.float32),
        pltpu.SemaphoreType.DMA((2,)),
        pltpu.SemaphoreType.DMA((2,)),
    ],
)


def kernel(time, table):
    idx_t = jnp.transpose(time)
    out_phys = _gather(idx_t, table.reshape(_VD))
    return jnp.transpose(out_phys, (2, 0, 1))


# speedup vs baseline: 5.2941x; 3.0885x over previous
import jax
import jax.numpy as jnp
from jax import lax
from jax.experimental import pallas as pl
from jax.experimental.pallas import tpu as pltpu
from jax.experimental.pallas import tpu_sc as plsc

_D = 64
_V = 49
_VD = _V * _D
_REP = 3137
_BATCH = 16384
_HIST = 200
_NC = 2
_NS = 16
_NW = _NC * _NS
_BB = 128
_TI = 8
_TO = 2
_NBLK = _BATCH // _BB // _NW
_NTC = _HIST // _TI
_NQ = _NBLK * _NTC


def _body(idx_hbm, table_hbm, out_hbm, stage_v, rep_v, idx_v0, idx_v1,
          rows_v0, rows_v1, idx_sem, out_sem):
    idx_bufs = (idx_v0, idx_v1)
    row_bufs = (rows_v0, rows_v1)
    wid = lax.axis_index("s") * _NC + lax.axis_index("c")
    wb0 = wid * (_NBLK * _BB)
    iota = lax.iota(jnp.int32, 16)
    lane_off = iota * _REP

    pltpu.sync_copy(table_hbm, stage_v)

    @plsc.parallel_loop(0, _VD // 16, step=1, unroll=2)
    def _build(m):
        x = stage_v[pl.ds(m * 16, 16)]
        dst = iota + m * 16
        for k in range(16):
            plsc.store_scatter(rep_v, [dst + k * _REP], x)

    def chunk_coords(q):
        blk = q // _NTC
        t0 = (q % _NTC) * _TI
        b0 = wb0 + blk * _BB
        return t0, b0

    def idx_start(q, qb):
        t0, b0 = chunk_coords(q)
        return pltpu.make_async_copy(
            idx_hbm.at[pl.ds(t0, _TI), pl.ds(b0, _BB)], idx_bufs[qb],
            idx_sem.at[qb])

    idx_start(0, 0).start()
    idx_start(1, 1).start()

    def compute(qb, h, rb):
        for tt in range(_TO):
            tl = h * _TO + tt
            pats = [idx_bufs[qb][tl, pl.ds(bg * 16, 16)] * _D + lane_off
                    for bg in range(_BB // 16)]

            @plsc.parallel_loop(0, _D, step=1, unroll=1)
            def j_step(j):
                for bg in range(_BB // 16):
                    x = plsc.load_gather(rep_v, [pats[bg] + j])
                    row_bufs[rb][tt, j, pl.ds(bg * 16, 16)] = x

    def q_step(q2, carry):
        for qb in range(2):
            q = q2 * 2 + qb
            t0, b0 = chunk_coords(q)
            idx_start(q, qb).wait()

            def h_step(h2, c2):
                for hh in range(2):
                    h = h2 * 2 + hh
                    rb = hh
                    dst = out_hbm.at[pl.ds(t0 + h * _TO, _TO), :,
                                     pl.ds(b0, _BB)]

                    def _wait_out():
                        pltpu.make_async_copy(row_bufs[rb], dst,
                                              out_sem.at[rb]).wait()

                    if qb == 0:
                        pl.when(jnp.logical_or(q2 >= 1, h2 >= 1))(_wait_out)
                    else:
                        _wait_out()

                    compute(qb, h, rb)
                    pltpu.async_copy(row_bufs[rb], dst, out_sem.at[rb])
                return c2

            lax.fori_loop(0, _TI // _TO // 2, h_step, 0)

            @pl.when(q2 < _NQ // 2 - 1)
            def _prefetch_idx():
                idx_start(q + 2, qb).start()
        return carry

    lax.fori_loop(0, _NQ // 2, q_step, 0)

    for rb in range(2):
        pltpu.make_async_copy(
            row_bufs[rb],
            out_hbm.at[pl.ds(0, _TO), :, pl.ds(wb0, _BB)],
            out_sem.at[rb]).wait()


_mesh = plsc.VectorSubcoreMesh(core_axis_name="c", subcore_axis_name="s")

_gather = pl.kernel(
    _body,
    out_type=jax.ShapeDtypeStruct((_HIST, _D, _BATCH), jnp.float32),
    mesh=_mesh,
    compiler_params=pltpu.CompilerParams(needs_layout_passes=False),
    scratch_types=[
        pltpu.VMEM((_VD,), jnp.float32),
        pltpu.VMEM((16 * _REP,), jnp.float32),
        pltpu.VMEM((_TI, _BB), jnp.int32),
        pltpu.VMEM((_TI, _BB), jnp.int32),
        pltpu.VMEM((_TO, _D, _BB), jnp.float32),
        pltpu.VMEM((_TO, _D, _BB), jnp.float32),
        pltpu.SemaphoreType.DMA((2,)),
        pltpu.SemaphoreType.DMA((2,)),
    ],
)


def kernel(time, table):
    idx_t = jnp.transpose(time)
    out_phys = _gather(idx_t, table.reshape(_VD))
    return jnp.transpose(out_phys, (2, 0, 1))
